# baseline (speedup 1.0000x reference)
import functools
import math

import jax
import jax.numpy as jnp
from jax import lax
from jax.experimental import pallas as pl
from jax.experimental.pallas import tpu as pltpu
from jax.experimental.pallas import tpu_sc as plsc

D_ = 1024
F_ = 2048
E_ = 8
K_ = 2
T_ = 2048
A_ = T_ * K_
B_ = 256
NB_ = 23
NPAD_ = NB_ * B_
BTC_ = 256

NW_ = 32
CH_ = 32


def _gelu_exact(x):
    return 0.5 * x * (1.0 + lax.erf(x * (1.0 / math.sqrt(2.0))))


def _router_meta_body(x_ref, w_ref, b_ref,
                      pos0_ref, pos1_ref, g0_ref, g1_ref, bexp_ref, aux_ref):
    x = x_ref[...]
    logits = jnp.dot(x, w_ref[...], preferred_element_type=jnp.float32)
    logits = logits + b_ref[...]
    m = jnp.max(logits, axis=1, keepdims=True)
    p = jnp.exp(logits - m)
    probs = p / jnp.sum(p, axis=1, keepdims=True)

    iota = lax.broadcasted_iota(jnp.int32, probs.shape, 1)
    m1 = jnp.max(probs, axis=1, keepdims=True)
    i1 = jnp.min(jnp.where(probs == m1, iota, E_), axis=1, keepdims=True)
    probs_wo1 = jnp.where(iota == i1, -1.0, probs)
    m2 = jnp.max(probs_wo1, axis=1, keepdims=True)
    i2 = jnp.min(jnp.where(probs_wo1 == m2, iota, E_), axis=1, keepdims=True)

    den = jnp.clip(m1 + m2, 1e-9, None)
    g0_ref[...] = m1 / den
    g1_ref[...] = m2 / den
    oh0 = (iota == i1).astype(jnp.float32)
    oh1 = (iota == i2).astype(jnp.float32)

    importance = jnp.sum(probs, axis=0) / float(T_)
    load = jnp.sum(oh0 + oh1, axis=0) / float(T_ * K_)
    aux = float(E_) * jnp.sum(importance * load)
    aux_ref[...] = jnp.full((1, 1), aux, dtype=jnp.float32)

    r = lax.broadcasted_iota(jnp.int32, (BTC_, BTC_), 0)
    c = lax.broadcasted_iota(jnp.int32, (BTC_, BTC_), 1)
    tril = (r >= c).astype(jnp.float32)

    carry = jnp.zeros((1, E_), dtype=jnp.float32)
    cums = []
    for oh in (oh0, oh1):
        blocks = []
        for bi in range(T_ // BTC_):
            blk = oh[bi * BTC_:(bi + 1) * BTC_, :]
            inc = jnp.dot(tril, blk, preferred_element_type=jnp.float32)
            inc = inc + carry
            blocks.append(inc)
            carry = inc[BTC_ - 1:BTC_, :]
        cums.append(jnp.concatenate(blocks, axis=0))
    cum0, cum1 = cums
    totals = carry

    pc = jnp.floor((totals + float(B_ - 1)) * (1.0 / B_)) * float(B_)
    er = lax.broadcasted_iota(jnp.int32, (E_, E_), 0)
    ec = lax.broadcasted_iota(jnp.int32, (E_, E_), 1)
    ustrict = (er < ec).astype(jnp.float32)
    base = jnp.dot(pc, ustrict, preferred_element_type=jnp.float32)
    ends = base + pc

    pos0 = jnp.sum(oh0 * (base + cum0 - 1.0), axis=1, keepdims=True)
    pos1 = jnp.sum(oh1 * (base + cum1 - 1.0), axis=1, keepdims=True)
    pos0_ref[...] = pos0.astype(jnp.int32)
    pos1_ref[...] = pos1.astype(jnp.int32)

    bstart = lax.broadcasted_iota(
        jnp.int32, (32, E_), 0).astype(jnp.float32) * float(B_)
    ge = (bstart >= ends).astype(jnp.float32)
    be = jnp.sum(ge, axis=1, keepdims=True)
    bexp_ref[...] = jnp.minimum(be, float(E_ - 1)).astype(jnp.int32)


def _run_router(flat, router_W, router_b):
    return pl.pallas_call(
        _router_meta_body,
        out_shape=(
            jax.ShapeDtypeStruct((T_, 1), jnp.int32),
            jax.ShapeDtypeStruct((T_, 1), jnp.int32),
            jax.ShapeDtypeStruct((T_, 1), jnp.float32),
            jax.ShapeDtypeStruct((T_, 1), jnp.float32),
            jax.ShapeDtypeStruct((32, 1), jnp.int32),
            jax.ShapeDtypeStruct((1, 1), jnp.float32),
        ),
    )(flat, router_W, router_b.reshape(1, E_))


def _dispatch_body(x_hbm, posk_hbm, gk_hbm, xs_hbm, gs_hbm,
                   idx_v, xbuf_a, xbuf_b, gbuf, semxa, semxb, sems, semg):
    cid = lax.axis_index("c")
    sid = lax.axis_index("s")
    wid = sid * 2 + cid
    t0 = wid * (T_ // NW_)
    b0 = t0
    b1 = t0 + CH_
    pltpu.sync_copy(posk_hbm.at[0, pl.ds(b0, CH_)], idx_v.at[0])
    pltpu.sync_copy(posk_hbm.at[1, pl.ds(b0, CH_)], idx_v.at[1])
    pltpu.sync_copy(posk_hbm.at[0, pl.ds(b1, CH_)], idx_v.at[2])
    pltpu.sync_copy(posk_hbm.at[1, pl.ds(b1, CH_)], idx_v.at[3])
    pltpu.sync_copy(gk_hbm.at[0, pl.ds(b0, CH_)], gbuf.at[0])
    pltpu.sync_copy(gk_hbm.at[1, pl.ds(b0, CH_)], gbuf.at[1])
    pltpu.sync_copy(gk_hbm.at[0, pl.ds(b1, CH_)], gbuf.at[2])
    pltpu.sync_copy(gk_hbm.at[1, pl.ds(b1, CH_)], gbuf.at[3])
    cpa = pltpu.async_copy(x_hbm.at[pl.ds(b0, CH_)], xbuf_a, semxa)
    cpb = pltpu.async_copy(x_hbm.at[pl.ds(b1, CH_)], xbuf_b, semxb)
    cpa.wait()
    waits = []
    for j in (0, 1):
        waits.append(pltpu.async_copy(xbuf_a, xs_hbm.at[idx_v.at[j]], sems))
        waits.append(pltpu.async_copy(gbuf.at[j], gs_hbm.at[idx_v.at[j]],
                                      semg))
    cpb.wait()
    for j in (2, 3):
        waits.append(pltpu.async_copy(xbuf_b, xs_hbm.at[idx_v.at[j]], sems))
        waits.append(pltpu.async_copy(gbuf.at[j], gs_hbm.at[idx_v.at[j]],
                                      semg))
    for w in waits:
        w.wait()


def _run_dispatch(flat, posk, gk):
    mesh = plsc.VectorSubcoreMesh(core_axis_name="c", subcore_axis_name="s")
    return pl.kernel(
        _dispatch_body,
        out_type=(
            jax.ShapeDtypeStruct((NPAD_, D_), jnp.float32),
            jax.ShapeDtypeStruct((NPAD_,), jnp.float32),
        ),
        mesh=mesh,
        scratch_types=[
            pltpu.VMEM((4, CH_), jnp.int32),
            pltpu.VMEM((CH_, D_), jnp.float32),
            pltpu.VMEM((CH_, D_), jnp.float32),
            pltpu.VMEM((4, CH_), jnp.float32),
            pltpu.SemaphoreType.DMA,
            pltpu.SemaphoreType.DMA,
            pltpu.SemaphoreType.DMA,
            pltpu.SemaphoreType.DMA,
        ],
    )(flat, posk, gk)


def _ffn_body(bexp_sref, x_ref, w1_ref, b1_ref, w2_ref, b2_ref, gs_ref,
              out_ref):
    del bexp_sref
    xb = x_ref[...].astype(jnp.bfloat16)
    h = jnp.dot(xb, w1_ref[0], preferred_element_type=jnp.float32)
    h = _gelu_exact(h + b1_ref[0])
    y = jnp.dot(h.astype(jnp.bfloat16), w2_ref[0],
                preferred_element_type=jnp.float32)
    out_ref[...] = (y + b2_ref[0]) * gs_ref[...]


def _run_ffn(xs, W1, b1, W2, b2, gs, bexp):
    grid_spec = pltpu.PrefetchScalarGridSpec(
        num_scalar_prefetch=1,
        grid=(NB_,),
        in_specs=[
            pl.BlockSpec((B_, D_), lambda i, be: (i, 0)),
            pl.BlockSpec((1, D_, F_), lambda i, be: (be[i], 0, 0)),
            pl.BlockSpec((1, 1, F_), lambda i, be: (be[i], 0, 0)),
            pl.BlockSpec((1, F_, D_), lambda i, be: (be[i], 0, 0)),
            pl.BlockSpec((1, 1, D_), lambda i, be: (be[i], 0, 0)),
            pl.BlockSpec((B_, 1), lambda i, be: (i, 0)),
        ],
        out_specs=pl.BlockSpec((B_, D_), lambda i, be: (i, 0)),
    )
    return pl.pallas_call(
        _ffn_body,
        grid_spec=grid_spec,
        out_shape=jax.ShapeDtypeStruct((NPAD_, D_), jnp.float32),
    )(bexp, xs, W1.astype(jnp.bfloat16), b1.reshape(E_, 1, F_),
      W2.astype(jnp.bfloat16), b2.reshape(E_, 1, D_), gs.reshape(NPAD_, 1))


def _combine_body(ys_hbm, posk_hbm, out_hbm,
                  idx0, idx1, buf0, buf1, sem_a, sem_b):
    cid = lax.axis_index("c")
    sid = lax.axis_index("s")
    wid = sid * 2 + cid
    t0 = wid * (T_ // NW_)
    for ci in range(T_ // NW_ // CH_):
        b = t0 + ci * CH_
        pltpu.sync_copy(posk_hbm.at[0, pl.ds(b, CH_)], idx0)
        pltpu.sync_copy(posk_hbm.at[1, pl.ds(b, CH_)], idx1)
        ca = pltpu.async_copy(ys_hbm.at[idx0], buf0, sem_a)
        cb = pltpu.async_copy(ys_hbm.at[idx1], buf1, sem_b)
        ca.wait()
        cb.wait()

        def body(j, carry):
            for sseg in range(D_ // 16):
                sl = pl.ds(sseg * 16, 16)
                buf0[j, sl] = buf0[j, sl] + buf1[j, sl]
            return carry

        lax.fori_loop(0, CH_, body, 0)
        pltpu.sync_copy(buf0, out_hbm.at[pl.ds(b, CH_)])


def _run_combine(ys, posk):
    mesh = plsc.VectorSubcoreMesh(core_axis_name="c", subcore_axis_name="s")
    return pl.kernel(
        _combine_body,
        out_type=jax.ShapeDtypeStruct((T_, D_), jnp.float32),
        mesh=mesh,
        scratch_types=[
            pltpu.VMEM((CH_,), jnp.int32),
            pltpu.VMEM((CH_,), jnp.int32),
            pltpu.VMEM((CH_, D_), jnp.float32),
            pltpu.VMEM((CH_, D_), jnp.float32),
            pltpu.SemaphoreType.DMA,
            pltpu.SemaphoreType.DMA,
        ],
    )(ys, posk)


def kernel(x, router_W, router_b, W1, b1, W2, b2):
    orig_shape = x.shape
    flat = x.reshape(-1, orig_shape[-1])

    pos0, pos1, g0, g1, bexp, aux = _run_router(flat, router_W, router_b)
    posk = jnp.concatenate([pos0.reshape(1, T_), pos1.reshape(1, T_)], axis=0)
    gk = jnp.concatenate([g0.reshape(1, T_), g1.reshape(1, T_)], axis=0)

    xs, gs = _run_dispatch(flat, posk, gk)
    ys = _run_ffn(xs, W1, b1, W2, b2, gs, bexp.reshape(-1))
    out = _run_combine(ys, posk)

    return out.reshape(orig_shape), aux.reshape(())
